# baseline (speedup 1.0000x reference)
import functools

import jax
import jax.numpy as jnp
from jax import lax
from jax.experimental import pallas as pl
from jax.experimental.pallas import tpu as pltpu
from jax.experimental.pallas import tpu_sc as plsc

N = 10000
E = 320000
D = 128
NA = 8
NB = 8
MG = 64

BN = 400
BE = 2000

NC = 2
NS = 16
NW = NC * NS

_INV_SQRT_D = 1.0 / (D ** 0.5)
_INV_SQRT_NB = 1.0 / (NB ** 0.5)
_INV_SQRT_DNA = 1.0 / ((D * NA) ** 0.5)


def _pre_body(x_ref, attrs_ref, wlin1_ref, wsc_ref, x1_ref, scs_ref):
    x = x_ref[...]
    x1_ref[...] = jnp.dot(x, wlin1_ref[...],
                          preferred_element_type=jnp.float32) * _INV_SQRT_D
    attrs = attrs_ref[...]
    acc = jnp.zeros((BN, D + MG), jnp.float32)
    for v in range(NA):
        acc = acc + jnp.dot(x * attrs[:, v:v + 1], wsc_ref[v],
                            preferred_element_type=jnp.float32)
    scs_ref[...] = acc * _INV_SQRT_DNA


def _msg_body(ee_ref, ea_ref, xi_ref, wfc1_ref, wfc2_ref, out_ref):
    ee = ee_ref[...]
    h = jnp.dot(ee, wfc1_ref[...], preferred_element_type=jnp.float32) * _INV_SQRT_NB
    h = h * jax.nn.sigmoid(h)
    w = jnp.dot(h, wfc2_ref[...], preferred_element_type=jnp.float32) * _INV_SQRT_NB
    xi = xi_ref[...]
    w0 = w[:, :D]
    w1 = w[:, D:]
    ea = ea_ref[...]
    out_ref[:, 0:D] = xi * w0 * ea[:, 0:1]
    xw = xi * w1
    for c in range(3):
        out_ref[:, D * (c + 1):D * (c + 2)] = xw * ea[:, c + 1:c + 2]


def _post_body(xout_ref, scs_ref, wl2s_ref, wl2v_ref, out_ref):
    s_out = jnp.dot(xout_ref[:, :D], wl2s_ref[...],
                    preferred_element_type=jnp.float32) * _INV_SQRT_D
    feat_s = s_out + scs_ref[...]
    scalars = feat_s[:, :D]
    out_ref[:, :D] = scalars * jax.nn.sigmoid(scalars)
    gates = feat_s[:, D:D + MG]
    gates = gates * jax.nn.sigmoid(gates)
    for c in range(3):
        v_c = jnp.dot(xout_ref[:, D * (c + 1):D * (c + 2)], wl2v_ref[...],
                      preferred_element_type=jnp.float32) * _INV_SQRT_D
        out_ref[:, D + MG * c:D + MG * (c + 1)] = v_c * gates


NP = 10240
NPS = NP // NS

_SC_MESH = plsc.VectorSubcoreMesh(core_axis_name="c", subcore_axis_name="s")

EH = E // 2


def _make_gather(n_edges, kg):
    gepw = n_edges // NW
    nbg = gepw // kg
    assert gepw % kg == 0 and kg % 8 == 0 and gepw % 8 == 0

    @functools.partial(
        pl.kernel,
        out_type=jax.ShapeDtypeStruct((n_edges, D), jnp.float32),
        mesh=_SC_MESH,
        scratch_types=[
            pltpu.VMEM((kg,), jnp.int32),
            pltpu.VMEM((kg,), jnp.int32),
            pltpu.VMEM((kg, D), jnp.float32),
            pltpu.VMEM((kg, D), jnp.float32),
            pltpu.SemaphoreType.DMA,
            pltpu.SemaphoreType.DMA,
            pltpu.SemaphoreType.DMA,
            pltpu.SemaphoreType.DMA,
        ],
    )
    def gather_sc(x1_hbm, idx_hbm, out_hbm, idx0, idx1, rows0, rows1,
                  gsem0, gsem1, ssem0, ssem1):
        wid = lax.axis_index("s") * NC + lax.axis_index("c")
        base = wid * gepw
        idxs = (idx0, idx1)
        rows = (rows0, rows1)
        gsems = (gsem0, gsem1)
        ssems = (ssem0, ssem1)

        for b in range(2):
            off = pl.multiple_of(base + b * kg, kg)
            pltpu.sync_copy(idx_hbm.at[pl.ds(off, kg)], idxs[b])
            pltpu.async_copy(x1_hbm.at[idxs[b]], rows[b], gsems[b])

        def body(gg, carry):
            for b in range(2):
                g = gg * 2 + b
                off = pl.multiple_of(base + g * kg, kg)
                pltpu.make_async_copy(x1_hbm.at[idxs[b]], rows[b],
                                      gsems[b]).wait()
                pltpu.async_copy(rows[b], out_hbm.at[pl.ds(off, kg)],
                                 ssems[b])

                @pl.when(g + 2 < nbg)
                def _prep():
                    off2 = pl.multiple_of(base + (g + 2) * kg, kg)
                    pltpu.make_async_copy(rows[b],
                                          out_hbm.at[pl.ds(off, kg)],
                                          ssems[b]).wait()
                    pltpu.sync_copy(idx_hbm.at[pl.ds(off2, kg)], idxs[b])
                    pltpu.async_copy(x1_hbm.at[idxs[b]], rows[b], gsems[b])
            return carry

        lax.fori_loop(0, (nbg - 1) // 2, body, 0)
        if nbg % 2 == 1:
            off = pl.multiple_of(base + (nbg - 1) * kg, kg)
            pltpu.make_async_copy(x1_hbm.at[idxs[0]], rows[0],
                                  gsems[0]).wait()
            pltpu.async_copy(rows[0], out_hbm.at[pl.ds(off, kg)], ssems[0])
        for g in (nbg - 2, nbg - 1):
            b = g % 2
            off = pl.multiple_of(base + g * kg, kg)
            pltpu.make_async_copy(rows[b], out_hbm.at[pl.ds(off, kg)],
                                  ssems[b]).wait()

    return gather_sc


def _make_scatter(n_edges, ks):
    eps = n_edges // NS
    nbs = eps // ks
    assert eps % ks == 0 and ks % 8 == 0 and nbs % 2 == 0

    @functools.partial(
        pl.kernel,
        out_type=jax.ShapeDtypeStruct((NP, 4 * D), jnp.float32),
        mesh=_SC_MESH,
        scratch_types=[
            pltpu.VMEM((1, ks), jnp.int32),
            pltpu.VMEM((1, ks), jnp.int32),
            pltpu.VMEM((ks, D), jnp.float32),
            pltpu.VMEM((ks, D), jnp.float32),
            pltpu.SemaphoreType.DMA,
            pltpu.SemaphoreType.DMA,
            pltpu.SemaphoreType.DMA,
            pltpu.SemaphoreType.DMA,
            pltpu.VMEM_SHARED((NP, D), jnp.float32),
        ],
    )
    def scatter_sc(msgs_hbm, ejg_hbm, init_hbm, out_hbm, idx0, idx1,
                   rows0, rows1, msem0, msem1, asem0, asem1, acc):
        core = lax.axis_index("c")
        sid = lax.axis_index("s")
        idxs = (idx0, idx1)
        rows = (rows0, rows1)
        msems = (msem0, msem1)
        asems = (asem0, asem1)
        for r in range(4 // NC):
            chunk = core * (4 // NC) + r
            col0 = pl.multiple_of(chunk * D, D)
            row0 = pl.multiple_of(sid * NPS, NPS)
            pltpu.sync_copy(init_hbm.at[pl.ds(row0, NPS), pl.ds(col0, D)],
                            acc.at[pl.ds(row0, NPS)])
            plsc.subcore_barrier()

            for b in range(2):
                blk = sid * nbs + b
                e0 = pl.multiple_of(sid * eps + b * ks, ks)
                pltpu.sync_copy(ejg_hbm.at[pl.ds(blk, 1)], idxs[b])
                pltpu.async_copy(msgs_hbm.at[pl.ds(e0, ks), pl.ds(col0, D)],
                                 rows[b], msems[b])

            def body(gg, carry):
                for b in range(2):
                    g = gg * 2 + b
                    e0 = pl.multiple_of(sid * eps + g * ks, ks)
                    e2 = pl.multiple_of(sid * eps + (g + 2) * ks, ks)
                    blk2 = sid * nbs + g + 2
                    pltpu.make_async_copy(
                        msgs_hbm.at[pl.ds(e0, ks), pl.ds(col0, D)],
                        rows[b], msems[b]).wait()
                    pltpu.async_copy(rows[b], acc.at[idxs[b].at[0]],
                                     asems[b], add=True)
                    pltpu.make_async_copy(rows[b], acc.at[idxs[b].at[0]],
                                          asems[b]).wait()
                    pltpu.sync_copy(ejg_hbm.at[pl.ds(blk2, 1)], idxs[b])
                    pltpu.async_copy(
                        msgs_hbm.at[pl.ds(e2, ks), pl.ds(col0, D)],
                        rows[b], msems[b])
                return carry

            lax.fori_loop(0, nbs // 2 - 1, body, 0)
            for b in range(2):
                g = nbs - 2 + b
                e0 = pl.multiple_of(sid * eps + g * ks, ks)
                pltpu.make_async_copy(
                    msgs_hbm.at[pl.ds(e0, ks), pl.ds(col0, D)],
                    rows[b], msems[b]).wait()
                pltpu.async_copy(rows[b], acc.at[idxs[b].at[0]], asems[b],
                                 add=True)
            for b in range(2):
                pltpu.make_async_copy(rows[b], acc.at[idxs[b].at[0]],
                                      asems[b]).wait()
            plsc.subcore_barrier()
            pltpu.sync_copy(acc.at[pl.ds(row0, NPS)],
                            out_hbm.at[pl.ds(row0, NPS), pl.ds(col0, D)])
            plsc.subcore_barrier()

    return scatter_sc


KS = 40
_gather_half = _make_gather(EH, 40)
_scatter_half = _make_scatter(EH, KS)


def _full(shape):
    ndim = len(shape)
    return pl.BlockSpec(shape, lambda i, _n=ndim: (0,) * _n)


def kernel(node_features, node_attrs, edge_index, edge_attrs, edge_embedding,
           W_lin1, W_fc1, W_fc2, W_lin2_s, W_lin2_v, W_sc):
    wsc_t = W_sc.transpose(1, 0, 2)
    x1, sc_s = pl.pallas_call(
        _pre_body,
        grid=(N // BN,),
        in_specs=[
            pl.BlockSpec((BN, D), lambda i: (i, 0)),
            pl.BlockSpec((BN, NA), lambda i: (i, 0)),
            _full((D, D)),
            _full((NA, D, D + MG)),
        ],
        out_specs=[
            pl.BlockSpec((BN, D), lambda i: (i, 0)),
            pl.BlockSpec((BN, D + MG), lambda i: (i, 0)),
        ],
        out_shape=[
            jax.ShapeDtypeStruct((N, D), jnp.float32),
            jax.ShapeDtypeStruct((N, D + MG), jnp.float32),
        ],
    )(node_features, node_attrs, W_lin1, wsc_t)

    edge_i = edge_index[0]
    edge_j = edge_index[1]

    def msg_half(ee, ea, xi):
        return pl.pallas_call(
            _msg_body,
            grid=(EH // BE,),
            in_specs=[
                pl.BlockSpec((BE, NB), lambda i: (i, 0)),
                pl.BlockSpec((BE, 4), lambda i: (i, 0)),
                pl.BlockSpec((BE, D), lambda i: (i, 0)),
                _full((NB, NB)),
                _full((NB, 2 * D)),
            ],
            out_specs=pl.BlockSpec((BE, 4 * D), lambda i: (i, 0)),
            out_shape=jax.ShapeDtypeStruct((EH, 4 * D), jnp.float32),
        )(ee, ea, xi, W_fc1, W_fc2)

    x_i0 = _gather_half(x1, edge_i[:EH])
    x_i1 = _gather_half(x1, edge_i[EH:])
    msgs0 = msg_half(edge_embedding[:EH], edge_attrs[:EH], x_i0)
    msgs1 = msg_half(edge_embedding[EH:], edge_attrs[EH:], x_i1)
    zeros = jnp.zeros((NP, 4 * D), jnp.float32)
    x_out0 = _scatter_half(msgs0, edge_j[:EH].reshape(EH // KS, KS), zeros)
    x_out = _scatter_half(msgs1, edge_j[EH:].reshape(EH // KS, KS), x_out0)

    out_c = pl.pallas_call(
        _post_body,
        grid=(N // BN,),
        in_specs=[
            pl.BlockSpec((BN, 4 * D), lambda i: (i, 0)),
            pl.BlockSpec((BN, D + MG), lambda i: (i, 0)),
            _full((D, D + MG)),
            _full((D, MG)),
        ],
        out_specs=pl.BlockSpec((BN, D + 3 * MG), lambda i: (i, 0)),
        out_shape=jax.ShapeDtypeStruct((N, D + 3 * MG), jnp.float32),
    )(x_out, sc_s, W_lin2_s, W_lin2_v)

    scalars = out_c[:, :D]
    gated = out_c[:, D:].reshape(N, 3, MG).transpose(0, 2, 1).reshape(N, 3 * MG)
    return jnp.concatenate([scalars, gated], axis=1)


# speedup vs baseline: 1.0198x; 1.0198x over previous
import functools

import jax
import jax.numpy as jnp
from jax import lax
from jax.experimental import pallas as pl
from jax.experimental.pallas import tpu as pltpu
from jax.experimental.pallas import tpu_sc as plsc

N = 10000
E = 320000
D = 128
NA = 8
NB = 8
MG = 64

BN = 400
BE = 2000

NC = 2
NS = 16
NW = NC * NS

_INV_SQRT_D = 1.0 / (D ** 0.5)
_INV_SQRT_NB = 1.0 / (NB ** 0.5)
_INV_SQRT_DNA = 1.0 / ((D * NA) ** 0.5)


def _pre_body(x_ref, attrs_ref, wlin1_ref, wsc_ref, x1_ref, scs_ref):
    x = x_ref[...]
    x1_ref[...] = jnp.dot(x, wlin1_ref[...],
                          preferred_element_type=jnp.float32) * _INV_SQRT_D
    attrs = attrs_ref[...]
    acc = jnp.zeros((BN, D + MG), jnp.float32)
    for v in range(NA):
        acc = acc + jnp.dot(x * attrs[:, v:v + 1], wsc_ref[v],
                            preferred_element_type=jnp.float32)
    scs_ref[...] = acc * _INV_SQRT_DNA


def _msg_body(ee_ref, ea_ref, xi_ref, wfc1_ref, wfc2_ref, out_ref):
    ee = ee_ref[...]
    h = jnp.dot(ee, wfc1_ref[...], preferred_element_type=jnp.float32) * _INV_SQRT_NB
    h = h * jax.nn.sigmoid(h)
    w = jnp.dot(h, wfc2_ref[...], preferred_element_type=jnp.float32) * _INV_SQRT_NB
    xi = xi_ref[...]
    w0 = w[:, :D]
    w1 = w[:, D:]
    ea = ea_ref[...]
    out_ref[:, 0:D] = xi * w0 * ea[:, 0:1]
    xw = xi * w1
    for c in range(3):
        out_ref[:, D * (c + 1):D * (c + 2)] = xw * ea[:, c + 1:c + 2]


def _post_body(xout_ref, scs_ref, wl2s_ref, wl2v_ref, out_ref):
    s_out = jnp.dot(xout_ref[:, :D], wl2s_ref[...],
                    preferred_element_type=jnp.float32) * _INV_SQRT_D
    feat_s = s_out + scs_ref[...]
    scalars = feat_s[:, :D]
    out_ref[:, :D] = scalars * jax.nn.sigmoid(scalars)
    gates = feat_s[:, D:D + MG]
    gates = gates * jax.nn.sigmoid(gates)
    for c in range(3):
        v_c = jnp.dot(xout_ref[:, D * (c + 1):D * (c + 2)], wl2v_ref[...],
                      preferred_element_type=jnp.float32) * _INV_SQRT_D
        out_ref[:, D + MG * c:D + MG * (c + 1)] = v_c * gates


NP = 10240
NPS = NP // NS

_SC_MESH = plsc.VectorSubcoreMesh(core_axis_name="c", subcore_axis_name="s")

KG = 80
GEPW = E // NW
NBG = GEPW // KG

KS = 40
NRING = 5
EPS = E // NS
NBS = EPS // KS


@functools.partial(
    pl.kernel,
    out_type=jax.ShapeDtypeStruct((E, D), jnp.float32),
    mesh=_SC_MESH,
    scratch_types=[
        pltpu.VMEM((KG,), jnp.int32),
        pltpu.VMEM((KG,), jnp.int32),
        pltpu.VMEM((KG, D), jnp.float32),
        pltpu.VMEM((KG, D), jnp.float32),
        pltpu.SemaphoreType.DMA,
        pltpu.SemaphoreType.DMA,
        pltpu.SemaphoreType.DMA,
        pltpu.SemaphoreType.DMA,
    ],
)
def _gather_sc(x1_hbm, idx_hbm, out_hbm, idx0, idx1, rows0, rows1,
               gsem0, gsem1, ssem0, ssem1):
    wid = lax.axis_index("s") * NC + lax.axis_index("c")
    base = wid * GEPW
    idxs = (idx0, idx1)
    rows = (rows0, rows1)
    gsems = (gsem0, gsem1)
    ssems = (ssem0, ssem1)

    for b in range(2):
        off = pl.multiple_of(base + b * KG, KG)
        pltpu.sync_copy(idx_hbm.at[pl.ds(off, KG)], idxs[b])
        pltpu.async_copy(x1_hbm.at[idxs[b]], rows[b], gsems[b])

    def body(gg, carry):
        for b in range(2):
            g = gg * 2 + b
            off = pl.multiple_of(base + g * KG, KG)
            pltpu.make_async_copy(x1_hbm.at[idxs[b]], rows[b], gsems[b]).wait()
            pltpu.async_copy(rows[b], out_hbm.at[pl.ds(off, KG)], ssems[b])

            @pl.when(g + 2 < NBG)
            def _prep():
                off2 = pl.multiple_of(base + (g + 2) * KG, KG)
                pltpu.make_async_copy(rows[b], out_hbm.at[pl.ds(off, KG)],
                                      ssems[b]).wait()
                pltpu.sync_copy(idx_hbm.at[pl.ds(off2, KG)], idxs[b])
                pltpu.async_copy(x1_hbm.at[idxs[b]], rows[b], gsems[b])
        return carry

    lax.fori_loop(0, (NBG - 1) // 2, body, 0)
    off_last = pl.multiple_of(base + (NBG - 1) * KG, KG)
    pltpu.make_async_copy(x1_hbm.at[idxs[0]], rows[0], gsems[0]).wait()
    pltpu.async_copy(rows[0], out_hbm.at[pl.ds(off_last, KG)], ssems[0])
    off_m2 = pl.multiple_of(base + (NBG - 2) * KG, KG)
    pltpu.make_async_copy(rows[1], out_hbm.at[pl.ds(off_m2, KG)],
                          ssems[1]).wait()
    pltpu.make_async_copy(rows[0], out_hbm.at[pl.ds(off_last, KG)],
                          ssems[0]).wait()


@functools.partial(
    pl.kernel,
    out_type=jax.ShapeDtypeStruct((NP, 4 * D), jnp.float32),
    mesh=_SC_MESH,
    scratch_types=(
        [pltpu.VMEM((1, KS), jnp.int32) for _ in range(NRING)]
        + [pltpu.VMEM((KS, D), jnp.float32) for _ in range(NRING)]
        + [pltpu.SemaphoreType.DMA for _ in range(2 * NRING)]
        + [pltpu.VMEM_SHARED((NP, D), jnp.float32)]
    ),
)
def _scatter_sc(msgs_hbm, ejg_hbm, zeros_hbm, out_hbm, *scr):
    idxs = scr[0:NRING]
    rows = scr[NRING:2 * NRING]
    msems = scr[2 * NRING:3 * NRING]
    asems = scr[3 * NRING:4 * NRING]
    acc = scr[4 * NRING]
    core = lax.axis_index("c")
    sid = lax.axis_index("s")

    for r in range(4 // NC):
        chunk = core * (4 // NC) + r
        col0 = pl.multiple_of(chunk * D, D)
        row0 = pl.multiple_of(sid * NPS, NPS)
        pltpu.sync_copy(zeros_hbm.at[pl.ds(row0, NPS)],
                        acc.at[pl.ds(row0, NPS)])
        plsc.subcore_barrier()

        def stage_blk(g, b):
            blk = sid * NBS + g
            e0 = pl.multiple_of(sid * EPS + g * KS, KS)
            pltpu.sync_copy(ejg_hbm.at[pl.ds(blk, 1)], idxs[b])
            pltpu.async_copy(msgs_hbm.at[pl.ds(e0, KS), pl.ds(col0, D)],
                             rows[b], msems[b])

        def wait_msg(g, b):
            e0 = pl.multiple_of(sid * EPS + g * KS, KS)
            pltpu.make_async_copy(msgs_hbm.at[pl.ds(e0, KS), pl.ds(col0, D)],
                                  rows[b], msems[b]).wait()

        for b in range(2):
            stage_blk(b, b)

        def body(gg, carry):
            for u in range(NRING):
                g = gg * NRING + u
                wait_msg(g, u)
                pltpu.async_copy(rows[u], acc.at[idxs[u].at[0]], asems[u],
                                 add=True)

                @pl.when(g >= 3)
                def _drain():
                    ub = (u - 3) % NRING
                    pltpu.make_async_copy(rows[ub], acc.at[idxs[ub].at[0]],
                                          asems[ub]).wait()

                @pl.when(g + 2 < NBS)
                def _prep():
                    stage_blk(g + 2, (u + 2) % NRING)
            return carry

        lax.fori_loop(0, NBS // NRING, body, 0)
        for g in (NBS - 3, NBS - 2, NBS - 1):
            b = g % NRING
            pltpu.make_async_copy(rows[b], acc.at[idxs[b].at[0]],
                                  asems[b]).wait()
        plsc.subcore_barrier()
        pltpu.sync_copy(acc.at[pl.ds(row0, NPS)],
                        out_hbm.at[pl.ds(row0, NPS), pl.ds(col0, D)])
        plsc.subcore_barrier()


def _full(shape):
    ndim = len(shape)
    return pl.BlockSpec(shape, lambda i, _n=ndim: (0,) * _n)


def kernel(node_features, node_attrs, edge_index, edge_attrs, edge_embedding,
           W_lin1, W_fc1, W_fc2, W_lin2_s, W_lin2_v, W_sc):
    wsc_t = W_sc.transpose(1, 0, 2)
    x1, sc_s = pl.pallas_call(
        _pre_body,
        grid=(N // BN,),
        in_specs=[
            pl.BlockSpec((BN, D), lambda i: (i, 0)),
            pl.BlockSpec((BN, NA), lambda i: (i, 0)),
            _full((D, D)),
            _full((NA, D, D + MG)),
        ],
        out_specs=[
            pl.BlockSpec((BN, D), lambda i: (i, 0)),
            pl.BlockSpec((BN, D + MG), lambda i: (i, 0)),
        ],
        out_shape=[
            jax.ShapeDtypeStruct((N, D), jnp.float32),
            jax.ShapeDtypeStruct((N, D + MG), jnp.float32),
        ],
    )(node_features, node_attrs, W_lin1, wsc_t)

    x_i = _gather_sc(x1, edge_index[0])

    msgs = pl.pallas_call(
        _msg_body,
        grid=(E // BE,),
        in_specs=[
            pl.BlockSpec((BE, NB), lambda i: (i, 0)),
            pl.BlockSpec((BE, 4), lambda i: (i, 0)),
            pl.BlockSpec((BE, D), lambda i: (i, 0)),
            _full((NB, NB)),
            _full((NB, 2 * D)),
        ],
        out_specs=pl.BlockSpec((BE, 4 * D), lambda i: (i, 0)),
        out_shape=jax.ShapeDtypeStruct((E, 4 * D), jnp.float32),
    )(edge_embedding, edge_attrs, x_i, W_fc1, W_fc2)

    ejg = edge_index[1].reshape(E // KS, KS)
    zeros = jnp.zeros((NP, D), jnp.float32)
    x_out = _scatter_sc(msgs, ejg, zeros)

    out_c = pl.pallas_call(
        _post_body,
        grid=(N // BN,),
        in_specs=[
            pl.BlockSpec((BN, 4 * D), lambda i: (i, 0)),
            pl.BlockSpec((BN, D + MG), lambda i: (i, 0)),
            _full((D, D + MG)),
            _full((D, MG)),
        ],
        out_specs=pl.BlockSpec((BN, D + 3 * MG), lambda i: (i, 0)),
        out_shape=jax.ShapeDtypeStruct((N, D + 3 * MG), jnp.float32),
    )(x_out, sc_s, W_lin2_s, W_lin2_v)

    scalars = out_c[:, :D]
    gated = out_c[:, D:].reshape(N, 3, MG).transpose(0, 2, 1).reshape(N, 3 * MG)
    return jnp.concatenate([scalars, gated], axis=1)


# speedup vs baseline: 1.1761x; 1.1533x over previous
import functools

import jax
import jax.numpy as jnp
from jax import lax
from jax.experimental import pallas as pl
from jax.experimental.pallas import tpu as pltpu
from jax.experimental.pallas import tpu_sc as plsc

N = 10000
E = 320000
D = 128
NA = 8
NB = 8
MG = 64

BN = 1000
BE = 4000

NC = 2
NS = 16
NW = NC * NS

_INV_SQRT_D = 1.0 / (D ** 0.5)
_INV_SQRT_NB = 1.0 / (NB ** 0.5)
_INV_SQRT_DNA = 1.0 / ((D * NA) ** 0.5)


def _pre_body(x_ref, attrs_ref, wlin1_ref, wsc_ref, x1_ref, scs_ref):
    x = x_ref[...]
    x1_ref[...] = jnp.dot(x, wlin1_ref[...],
                          preferred_element_type=jnp.float32) * _INV_SQRT_D
    attrs = attrs_ref[...]
    acc = jnp.zeros((BN, D + MG), jnp.float32)
    for v in range(NA):
        acc = acc + jnp.dot(x * attrs[:, v:v + 1], wsc_ref[v],
                            preferred_element_type=jnp.float32)
    scs_ref[...] = acc * _INV_SQRT_DNA


def _msg_body(ee_ref, ea_ref, xi_ref, wfc1_ref, wfc2_ref, out_ref):
    ee = ee_ref[...]
    h = jnp.dot(ee, wfc1_ref[...], preferred_element_type=jnp.float32) * _INV_SQRT_NB
    h = h * jax.nn.sigmoid(h)
    w = jnp.dot(h, wfc2_ref[...], preferred_element_type=jnp.float32) * _INV_SQRT_NB
    xi = xi_ref[...]
    w0 = w[:, :D]
    w1 = w[:, D:]
    ea = ea_ref[...]
    out_ref[:, 0:D] = xi * w0 * ea[:, 0:1]
    xw = xi * w1
    for c in range(3):
        out_ref[:, D * (c + 1):D * (c + 2)] = xw * ea[:, c + 1:c + 2]


def _post_body(xout_ref, scs_ref, wl2s_ref, wl2v_ref, out_ref):
    s_out = jnp.dot(xout_ref[:, :D], wl2s_ref[...],
                    preferred_element_type=jnp.float32) * _INV_SQRT_D
    feat_s = s_out + scs_ref[...]
    scalars = feat_s[:, :D]
    out_ref[:, :D] = scalars * jax.nn.sigmoid(scalars)
    gates = feat_s[:, D:D + MG]
    gates = gates * jax.nn.sigmoid(gates)
    for c in range(3):
        v_c = jnp.dot(xout_ref[:, D * (c + 1):D * (c + 2)], wl2v_ref[...],
                      preferred_element_type=jnp.float32) * _INV_SQRT_D
        out_ref[:, D + MG * c:D + MG * (c + 1)] = v_c * gates


NP = 10240

_SC_MESH = plsc.VectorSubcoreMesh(core_axis_name="c", subcore_axis_name="s")

KG = 80
GEPW = E // NW
NBG = GEPW // KG

KS = 80
EPS = E // NS
NBS = EPS // KS
NPS = NP // NS


@functools.partial(
    pl.kernel,
    out_type=jax.ShapeDtypeStruct((E, D), jnp.float32),
    mesh=_SC_MESH,
    scratch_types=[
        pltpu.VMEM((KG,), jnp.int32),
        pltpu.VMEM((KG,), jnp.int32),
        pltpu.VMEM((KG, D), jnp.float32),
        pltpu.VMEM((KG, D), jnp.float32),
        pltpu.SemaphoreType.DMA,
        pltpu.SemaphoreType.DMA,
        pltpu.SemaphoreType.DMA,
        pltpu.SemaphoreType.DMA,
    ],
)
def _gather_sc(x1_hbm, idx_hbm, out_hbm, idx0, idx1, rows0, rows1,
               gsem0, gsem1, ssem0, ssem1):
    wid = lax.axis_index("s") * NC + lax.axis_index("c")
    base = wid * GEPW
    idxs = (idx0, idx1)
    rows = (rows0, rows1)
    gsems = (gsem0, gsem1)
    ssems = (ssem0, ssem1)

    for b in range(2):
        off = pl.multiple_of(base + b * KG, KG)
        pltpu.sync_copy(idx_hbm.at[pl.ds(off, KG)], idxs[b])
        pltpu.async_copy(x1_hbm.at[idxs[b]], rows[b], gsems[b])

    def body(gg, carry):
        for b in range(2):
            g = gg * 2 + b
            off = pl.multiple_of(base + g * KG, KG)
            pltpu.make_async_copy(x1_hbm.at[idxs[b]], rows[b], gsems[b]).wait()
            pltpu.async_copy(rows[b], out_hbm.at[pl.ds(off, KG)], ssems[b])

            @pl.when(g + 2 < NBG)
            def _prep():
                off2 = pl.multiple_of(base + (g + 2) * KG, KG)
                pltpu.make_async_copy(rows[b], out_hbm.at[pl.ds(off, KG)],
                                      ssems[b]).wait()
                pltpu.sync_copy(idx_hbm.at[pl.ds(off2, KG)], idxs[b])
                pltpu.async_copy(x1_hbm.at[idxs[b]], rows[b], gsems[b])
        return carry

    lax.fori_loop(0, (NBG - 1) // 2, body, 0)
    g_last = NBG - 1
    off = pl.multiple_of(base + g_last * KG, KG)
    pltpu.make_async_copy(x1_hbm.at[idxs[0]], rows[0], gsems[0]).wait()
    pltpu.async_copy(rows[0], out_hbm.at[pl.ds(off, KG)], ssems[0])
    off_m2 = pl.multiple_of(base + (NBG - 2) * KG, KG)
    pltpu.make_async_copy(rows[1], out_hbm.at[pl.ds(off_m2, KG)],
                          ssems[1]).wait()
    pltpu.make_async_copy(rows[0], out_hbm.at[pl.ds(off, KG)],
                          ssems[0]).wait()


@functools.partial(
    pl.kernel,
    out_type=jax.ShapeDtypeStruct((NP, 4 * D), jnp.float32),
    mesh=_SC_MESH,
    scratch_types=[
        pltpu.VMEM((1, KS), jnp.int32),
        pltpu.VMEM((1, KS), jnp.int32),
        pltpu.VMEM((KS, D), jnp.float32),
        pltpu.VMEM((KS, D), jnp.float32),
        pltpu.SemaphoreType.DMA,
        pltpu.SemaphoreType.DMA,
        pltpu.SemaphoreType.DMA,
        pltpu.SemaphoreType.DMA,
        pltpu.VMEM_SHARED((NP, D), jnp.float32),
    ],
)
def _scatter_sc(msgs_hbm, ejg_hbm, zeros_hbm, out_hbm, idx0, idx1,
                rows0, rows1, msem0, msem1, asem0, asem1, acc):
    core = lax.axis_index("c")
    sid = lax.axis_index("s")
    idxs = (idx0, idx1)
    rows = (rows0, rows1)
    msems = (msem0, msem1)
    asems = (asem0, asem1)
    for r in range(4 // NC):
        chunk = core * (4 // NC) + r
        col0 = pl.multiple_of(chunk * D, D)
        row0 = pl.multiple_of(sid * NPS, NPS)
        pltpu.sync_copy(zeros_hbm.at[pl.ds(row0, NPS)],
                        acc.at[pl.ds(row0, NPS)])
        plsc.subcore_barrier()

        for b in range(2):
            blk = sid * NBS + b
            e0 = pl.multiple_of(sid * EPS + b * KS, KS)
            pltpu.sync_copy(ejg_hbm.at[pl.ds(blk, 1)], idxs[b])
            pltpu.async_copy(msgs_hbm.at[pl.ds(e0, KS), pl.ds(col0, D)],
                             rows[b], msems[b])

        def body(gg, carry):
            for b in range(2):
                g = gg * 2 + b
                e0 = pl.multiple_of(sid * EPS + g * KS, KS)
                e2 = pl.multiple_of(sid * EPS + (g + 2) * KS, KS)
                blk2 = sid * NBS + g + 2
                pltpu.make_async_copy(
                    msgs_hbm.at[pl.ds(e0, KS), pl.ds(col0, D)],
                    rows[b], msems[b]).wait()
                pltpu.async_copy(rows[b], acc.at[idxs[b].at[0]], asems[b],
                                 add=True)
                pltpu.make_async_copy(rows[b], acc.at[idxs[b].at[0]],
                                      asems[b]).wait()
                pltpu.sync_copy(ejg_hbm.at[pl.ds(blk2, 1)], idxs[b])
                pltpu.async_copy(msgs_hbm.at[pl.ds(e2, KS), pl.ds(col0, D)],
                                 rows[b], msems[b])
            return carry

        lax.fori_loop(0, NBS // 2 - 1, body, 0)
        for b in range(2):
            g = NBS - 2 + b
            e0 = pl.multiple_of(sid * EPS + g * KS, KS)
            pltpu.make_async_copy(
                msgs_hbm.at[pl.ds(e0, KS), pl.ds(col0, D)],
                rows[b], msems[b]).wait()
            pltpu.async_copy(rows[b], acc.at[idxs[b].at[0]], asems[b],
                             add=True)
        for b in range(2):
            pltpu.make_async_copy(rows[b], acc.at[idxs[b].at[0]],
                                  asems[b]).wait()
        plsc.subcore_barrier()
        pltpu.sync_copy(acc.at[pl.ds(row0, NPS)],
                        out_hbm.at[pl.ds(row0, NPS), pl.ds(col0, D)])
        plsc.subcore_barrier()


def _full(shape):
    ndim = len(shape)
    return pl.BlockSpec(shape, lambda i, _n=ndim: (0,) * _n)


def kernel(node_features, node_attrs, edge_index, edge_attrs, edge_embedding,
           W_lin1, W_fc1, W_fc2, W_lin2_s, W_lin2_v, W_sc):
    wsc_t = W_sc.transpose(1, 0, 2)
    x1, sc_s = pl.pallas_call(
        _pre_body,
        grid=(N // BN,),
        in_specs=[
            pl.BlockSpec((BN, D), lambda i: (i, 0)),
            pl.BlockSpec((BN, NA), lambda i: (i, 0)),
            _full((D, D)),
            _full((NA, D, D + MG)),
        ],
        out_specs=[
            pl.BlockSpec((BN, D), lambda i: (i, 0)),
            pl.BlockSpec((BN, D + MG), lambda i: (i, 0)),
        ],
        out_shape=[
            jax.ShapeDtypeStruct((N, D), jnp.float32),
            jax.ShapeDtypeStruct((N, D + MG), jnp.float32),
        ],
    )(node_features, node_attrs, W_lin1, wsc_t)

    x_i = _gather_sc(x1, edge_index[0])

    msgs = pl.pallas_call(
        _msg_body,
        grid=(E // BE,),
        in_specs=[
            pl.BlockSpec((BE, NB), lambda i: (i, 0)),
            pl.BlockSpec((BE, 4), lambda i: (i, 0)),
            pl.BlockSpec((BE, D), lambda i: (i, 0)),
            _full((NB, NB)),
            _full((NB, 2 * D)),
        ],
        out_specs=pl.BlockSpec((BE, 4 * D), lambda i: (i, 0)),
        out_shape=jax.ShapeDtypeStruct((E, 4 * D), jnp.float32),
    )(edge_embedding, edge_attrs, x_i, W_fc1, W_fc2)

    ejg = edge_index[1].reshape(E // KS, KS)
    zeros = jnp.zeros((NP, D), jnp.float32)
    x_out = _scatter_sc(msgs, ejg, zeros)

    out_c = pl.pallas_call(
        _post_body,
        grid=(N // BN,),
        in_specs=[
            pl.BlockSpec((BN, 4 * D), lambda i: (i, 0)),
            pl.BlockSpec((BN, D + MG), lambda i: (i, 0)),
            _full((D, D + MG)),
            _full((D, MG)),
        ],
        out_specs=pl.BlockSpec((BN, D + 3 * MG), lambda i: (i, 0)),
        out_shape=jax.ShapeDtypeStruct((N, D + 3 * MG), jnp.float32),
    )(x_out, sc_s, W_lin2_s, W_lin2_v)

    scalars = out_c[:, :D]
    gated = out_c[:, D:].reshape(N, 3, MG).transpose(0, 2, 1).reshape(N, 3 * MG)
    return jnp.concatenate([scalars, gated], axis=1)


# speedup vs baseline: 1.2819x; 1.0900x over previous
import functools

import jax
import jax.numpy as jnp
from jax import lax
from jax.experimental import pallas as pl
from jax.experimental.pallas import tpu as pltpu
from jax.experimental.pallas import tpu_sc as plsc

N = 10000
E = 320000
D = 128
NA = 8
NB = 8
MG = 64

BN = 1000
BE = 4000

NC = 2
NS = 16
NW = NC * NS

_INV_SQRT_D = 1.0 / (D ** 0.5)
_INV_SQRT_NB = 1.0 / (NB ** 0.5)
_INV_SQRT_DNA = 1.0 / ((D * NA) ** 0.5)


def _pre_body(x_ref, attrs_ref, wlin1_ref, wsc_ref, x1_ref, scs_ref):
    x = x_ref[...]
    x1_ref[...] = jnp.dot(x, wlin1_ref[...],
                          preferred_element_type=jnp.float32) * _INV_SQRT_D
    attrs = attrs_ref[...]
    acc = jnp.zeros((BN, D + MG), jnp.float32)
    for v in range(NA):
        acc = acc + jnp.dot(x * attrs[:, v:v + 1], wsc_ref[v],
                            preferred_element_type=jnp.float32)
    scs_ref[...] = acc * _INV_SQRT_DNA


def _msg_body(ee_ref, ea_ref, xi_ref, wfc1_ref, wfc2_ref, out_ref):
    ee = ee_ref[...]
    h = jnp.dot(ee, wfc1_ref[...], preferred_element_type=jnp.float32) * _INV_SQRT_NB
    h = h * jax.nn.sigmoid(h)
    w = jnp.dot(h, wfc2_ref[...], preferred_element_type=jnp.float32) * _INV_SQRT_NB
    xi = xi_ref[...]
    w0 = w[:, :D]
    w1 = w[:, D:]
    ea = ea_ref[...]
    out_ref[:, 0:D] = xi * w0 * ea[:, 0:1]
    xw = xi * w1
    for c in range(3):
        out_ref[:, D * (c + 1):D * (c + 2)] = xw * ea[:, c + 1:c + 2]


def _post_body(xout_ref, scs_ref, wl2s_ref, wl2v_ref, out_ref):
    s_out = jnp.dot(xout_ref[:, :D], wl2s_ref[...],
                    preferred_element_type=jnp.float32) * _INV_SQRT_D
    feat_s = s_out + scs_ref[...]
    scalars = feat_s[:, :D]
    out_ref[:, :D] = scalars * jax.nn.sigmoid(scalars)
    gates = feat_s[:, D:D + MG]
    gates = gates * jax.nn.sigmoid(gates)
    for c in range(3):
        v_c = jnp.dot(xout_ref[:, D * (c + 1):D * (c + 2)], wl2v_ref[...],
                      preferred_element_type=jnp.float32) * _INV_SQRT_D
        out_ref[:, D + MG * c:D + MG * (c + 1)] = v_c * gates


NP = 10240

_SC_MESH = plsc.VectorSubcoreMesh(core_axis_name="c", subcore_axis_name="s")

KG = 80
GEPW = E // NW
NBG = GEPW // KG

KS = 80
EPS = E // NS
NBS = EPS // KS
NPS = NP // NS


@functools.partial(
    pl.kernel,
    out_type=jax.ShapeDtypeStruct((E, D), jnp.float32),
    mesh=_SC_MESH,
    scratch_types=[
        pltpu.VMEM((NBG, KG), jnp.int32),
        pltpu.VMEM((KG, D), jnp.float32),
        pltpu.VMEM((KG, D), jnp.float32),
        pltpu.SemaphoreType.DMA,
        pltpu.SemaphoreType.DMA,
        pltpu.SemaphoreType.DMA,
        pltpu.SemaphoreType.DMA,
    ],
)
def _gather_sc(x1_hbm, eig_hbm, out_hbm, idx_v, rows0, rows1,
               gsem0, gsem1, ssem0, ssem1):
    wid = lax.axis_index("s") * NC + lax.axis_index("c")
    base = wid * GEPW
    rows = (rows0, rows1)
    gsems = (gsem0, gsem1)
    ssems = (ssem0, ssem1)

    pltpu.sync_copy(eig_hbm.at[wid], idx_v)

    for b in range(2):
        pltpu.async_copy(x1_hbm.at[idx_v.at[b]], rows[b], gsems[b])

    def body(gg, carry):
        for b in range(2):
            g = gg * 2 + b
            off = pl.multiple_of(base + g * KG, KG)
            pltpu.make_async_copy(x1_hbm.at[idx_v.at[g]], rows[b],
                                  gsems[b]).wait()
            pltpu.async_copy(rows[b], out_hbm.at[pl.ds(off, KG)], ssems[b])

            @pl.when(g + 2 < NBG)
            def _prep():
                pltpu.make_async_copy(rows[b], out_hbm.at[pl.ds(off, KG)],
                                      ssems[b]).wait()
                pltpu.async_copy(x1_hbm.at[idx_v.at[g + 2]], rows[b],
                                 gsems[b])
        return carry

    lax.fori_loop(0, (NBG - 1) // 2, body, 0)
    g_last = NBG - 1
    off = pl.multiple_of(base + g_last * KG, KG)
    pltpu.make_async_copy(x1_hbm.at[idx_v.at[g_last]], rows[0],
                          gsems[0]).wait()
    pltpu.async_copy(rows[0], out_hbm.at[pl.ds(off, KG)], ssems[0])
    off_m2 = pl.multiple_of(base + (NBG - 2) * KG, KG)
    pltpu.make_async_copy(rows[1], out_hbm.at[pl.ds(off_m2, KG)],
                          ssems[1]).wait()
    pltpu.make_async_copy(rows[0], out_hbm.at[pl.ds(off, KG)],
                          ssems[0]).wait()


@functools.partial(
    pl.kernel,
    out_type=jax.ShapeDtypeStruct((NP, 4 * D), jnp.float32),
    mesh=_SC_MESH,
    scratch_types=[
        pltpu.VMEM((1, KS), jnp.int32),
        pltpu.VMEM((1, KS), jnp.int32),
        pltpu.VMEM((KS, D), jnp.float32),
        pltpu.VMEM((KS, D), jnp.float32),
        pltpu.SemaphoreType.DMA,
        pltpu.SemaphoreType.DMA,
        pltpu.SemaphoreType.DMA,
        pltpu.SemaphoreType.DMA,
        pltpu.SemaphoreType.DMA,
        pltpu.SemaphoreType.DMA,
        pltpu.VMEM_SHARED((NP, D), jnp.float32),
    ],
)
def _scatter_sc(msgs_hbm, ejg_hbm, zeros_hbm, out_hbm, idx0, idx1,
                rows0, rows1, msem0, msem1, asem0, asem1, isem0, isem1, acc):
    core = lax.axis_index("c")
    sid = lax.axis_index("s")
    idxs = (idx0, idx1)
    rows = (rows0, rows1)
    msems = (msem0, msem1)
    asems = (asem0, asem1)
    isems = (isem0, isem1)
    for r in range(4 // NC):
        chunk = core * (4 // NC) + r
        col0 = pl.multiple_of(chunk * D, D)
        row0 = pl.multiple_of(sid * NPS, NPS)
        pltpu.sync_copy(zeros_hbm.at[pl.ds(row0, NPS)],
                        acc.at[pl.ds(row0, NPS)])
        plsc.subcore_barrier()

        for b in range(2):
            blk = sid * NBS + b
            e0 = pl.multiple_of(sid * EPS + b * KS, KS)
            pltpu.async_copy(ejg_hbm.at[pl.ds(blk, 1)], idxs[b], isems[b])
            pltpu.async_copy(msgs_hbm.at[pl.ds(e0, KS), pl.ds(col0, D)],
                             rows[b], msems[b])

        def body(gg, carry):
            for b in range(2):
                g = gg * 2 + b
                blk = sid * NBS + g
                blk2 = sid * NBS + g + 2
                e0 = pl.multiple_of(sid * EPS + g * KS, KS)
                e2 = pl.multiple_of(sid * EPS + (g + 2) * KS, KS)
                pltpu.make_async_copy(ejg_hbm.at[pl.ds(blk, 1)], idxs[b],
                                      isems[b]).wait()
                pltpu.make_async_copy(
                    msgs_hbm.at[pl.ds(e0, KS), pl.ds(col0, D)],
                    rows[b], msems[b]).wait()
                pltpu.async_copy(rows[b], acc.at[idxs[b].at[0]], asems[b],
                                 add=True)
                pltpu.make_async_copy(rows[b], acc.at[idxs[b].at[0]],
                                      asems[b]).wait()
                pltpu.async_copy(ejg_hbm.at[pl.ds(blk2, 1)], idxs[b],
                                 isems[b])
                pltpu.async_copy(msgs_hbm.at[pl.ds(e2, KS), pl.ds(col0, D)],
                                 rows[b], msems[b])
            return carry

        lax.fori_loop(0, NBS // 2 - 1, body, 0)
        for b in range(2):
            g = NBS - 2 + b
            blk = sid * NBS + g
            e0 = pl.multiple_of(sid * EPS + g * KS, KS)
            pltpu.make_async_copy(ejg_hbm.at[pl.ds(blk, 1)], idxs[b],
                                  isems[b]).wait()
            pltpu.make_async_copy(
                msgs_hbm.at[pl.ds(e0, KS), pl.ds(col0, D)],
                rows[b], msems[b]).wait()
            pltpu.async_copy(rows[b], acc.at[idxs[b].at[0]], asems[b],
                             add=True)
        for b in range(2):
            pltpu.make_async_copy(rows[b], acc.at[idxs[b].at[0]],
                                  asems[b]).wait()
        plsc.subcore_barrier()
        pltpu.sync_copy(acc.at[pl.ds(row0, NPS)],
                        out_hbm.at[pl.ds(row0, NPS), pl.ds(col0, D)])
        plsc.subcore_barrier()


def _full(shape):
    ndim = len(shape)
    return pl.BlockSpec(shape, lambda i, _n=ndim: (0,) * _n)


def kernel(node_features, node_attrs, edge_index, edge_attrs, edge_embedding,
           W_lin1, W_fc1, W_fc2, W_lin2_s, W_lin2_v, W_sc):
    wsc_t = W_sc.transpose(1, 0, 2)
    x1, sc_s = pl.pallas_call(
        _pre_body,
        grid=(N // BN,),
        in_specs=[
            pl.BlockSpec((BN, D), lambda i: (i, 0)),
            pl.BlockSpec((BN, NA), lambda i: (i, 0)),
            _full((D, D)),
            _full((NA, D, D + MG)),
        ],
        out_specs=[
            pl.BlockSpec((BN, D), lambda i: (i, 0)),
            pl.BlockSpec((BN, D + MG), lambda i: (i, 0)),
        ],
        out_shape=[
            jax.ShapeDtypeStruct((N, D), jnp.float32),
            jax.ShapeDtypeStruct((N, D + MG), jnp.float32),
        ],
    )(node_features, node_attrs, W_lin1, wsc_t)

    eig = edge_index[0].reshape(NW, NBG, KG)
    x_i = _gather_sc(x1, eig)

    msgs = pl.pallas_call(
        _msg_body,
        grid=(E // BE,),
        in_specs=[
            pl.BlockSpec((BE, NB), lambda i: (i, 0)),
            pl.BlockSpec((BE, 4), lambda i: (i, 0)),
            pl.BlockSpec((BE, D), lambda i: (i, 0)),
            _full((NB, NB)),
            _full((NB, 2 * D)),
        ],
        out_specs=pl.BlockSpec((BE, 4 * D), lambda i: (i, 0)),
        out_shape=jax.ShapeDtypeStruct((E, 4 * D), jnp.float32),
    )(edge_embedding, edge_attrs, x_i, W_fc1, W_fc2)

    ejg = edge_index[1].reshape(E // KS, KS)
    zeros = jnp.zeros((NP, D), jnp.float32)
    x_out = _scatter_sc(msgs, ejg, zeros)

    out_c = pl.pallas_call(
        _post_body,
        grid=(N // BN,),
        in_specs=[
            pl.BlockSpec((BN, 4 * D), lambda i: (i, 0)),
            pl.BlockSpec((BN, D + MG), lambda i: (i, 0)),
            _full((D, D + MG)),
            _full((D, MG)),
        ],
        out_specs=pl.BlockSpec((BN, D + 3 * MG), lambda i: (i, 0)),
        out_shape=jax.ShapeDtypeStruct((N, D + 3 * MG), jnp.float32),
    )(x_out, sc_s, W_lin2_s, W_lin2_v)

    scalars = out_c[:, :D]
    gated = out_c[:, D:].reshape(N, 3, MG).transpose(0, 2, 1).reshape(N, 3 * MG)
    return jnp.concatenate([scalars, gated], axis=1)
